# Initial kernel scaffold; baseline (speedup 1.0000x reference)
import jax
import jax.numpy as jnp
from jax.experimental import pallas as pl


def kernel(pconf, pcls, pxywh):
    raise NotImplementedError("write your pallas kernel here")


# speedup vs baseline: 2.7759x; 2.7759x over previous
import functools

import jax
import jax.numpy as jnp
from jax.experimental import pallas as pl
from jax.experimental.pallas import tpu as pltpu

_CONF_THR = 0.05
_PTOPK = 500
_NMS_THR = 0.5
_K = 512


def _score_body(pconf_ref, pcls_ref, ps_ref, plab_ref):
    x = pcls_ref[0]
    s = jax.nn.sigmoid(x)
    m = jnp.max(s, axis=-1, keepdims=True)
    cidx = jax.lax.broadcasted_iota(jnp.int32, s.shape, 1)
    lab = jnp.min(jnp.where(s == m, cidx, jnp.int32(2**30)), axis=-1,
                  keepdims=True)
    conf = jax.nn.sigmoid(pconf_ref[0])
    ps_ref[0] = conf * m
    plab_ref[0] = lab


def _scores_labels(pconf, pcls):
    B, N, C = pcls.shape
    CHUNK = 2000
    grid = (B, N // CHUNK)
    ps, plab = pl.pallas_call(
        _score_body,
        grid=grid,
        in_specs=[
            pl.BlockSpec((1, CHUNK, 1), lambda b, c: (b, c, 0)),
            pl.BlockSpec((1, CHUNK, C), lambda b, c: (b, c, 0)),
        ],
        out_specs=[
            pl.BlockSpec((1, CHUNK, 1), lambda b, c: (b, c, 0)),
            pl.BlockSpec((1, CHUNK, 1), lambda b, c: (b, c, 0)),
        ],
        out_shape=[
            jax.ShapeDtypeStruct((B, N, 1), jnp.float32),
            jax.ShapeDtypeStruct((B, N, 1), jnp.int32),
        ],
        compiler_params=pltpu.CompilerParams(
            dimension_semantics=("arbitrary", "arbitrary"),
        ),
    )(pconf[..., None], pcls)
    return ps[..., 0], plab[..., 0]


def _sortiou_body(scores_ref, idx_ref, lab_ref,
                  bx_ref, by_ref, bw_ref, bh_ref,
                  b2_ref, l2_ref, s2m_ref, mask_ref):
    s = scores_ref[0, 0]
    ix = idx_ref[0, 0]
    lb = lab_ref[0, 0]

    T = 64
    nt = _K // T

    ranks = []
    for t in range(nt):
        si = s[t * T:(t + 1) * T][:, None]
        ixi = ix[t * T:(t + 1) * T][:, None]
        before = ((s[None, :] > si) |
                  ((s[None, :] == si) & (ix[None, :] < ixi)))
        ranks.append(jnp.sum(before.astype(jnp.int32), axis=1))
    rank = jnp.concatenate(ranks)

    cols = (bx_ref[0, 0], by_ref[0, 0], bw_ref[0, 0], bh_ref[0, 0], s)
    parts = [[] for _ in range(6)]
    for t in range(nt):
        ot_iota = jax.lax.broadcasted_iota(jnp.int32, (T, _K), 0) + t * T
        oh = (rank[None, :] == ot_iota)
        for ci, col in enumerate(cols):
            parts[ci].append(
                jnp.max(jnp.where(oh, col[None, :], jnp.float32(-3e38)),
                        axis=1))
        parts[5].append(
            jnp.max(jnp.where(oh, lb[None, :], jnp.int32(-2**31 + 1)),
                    axis=1))
    cx, cy, w, h, so = (jnp.concatenate(p) for p in parts[:5])
    lo = jnp.concatenate(parts[5])

    so = jnp.where(so > _CONF_THR, so, jnp.float32(0.0))

    xl = cx - w * 0.5
    yt = cy - h * 0.5
    xr = cx + w * 0.5
    yb = cy + h * 0.5

    b2_ref[0] = jnp.stack([xl, yt, xr, yb], axis=1)
    l2_ref[0, 0] = lo
    s2m_ref[0, 0] = so

    off = lo.astype(jnp.float32) * 4.0
    ol = xl + off
    ot = yt + off
    orr = xr + off
    ob = yb + off
    area = jnp.clip(orr - ol, 0.0) * jnp.clip(ob - ot, 0.0)
    for t in range(nt):
        sl = slice(t * T, (t + 1) * T)
        iw = jnp.clip(jnp.minimum(orr[sl][:, None], orr[None, :]) -
                      jnp.maximum(ol[sl][:, None], ol[None, :]), 0.0)
        ih = jnp.clip(jnp.minimum(ob[sl][:, None], ob[None, :]) -
                      jnp.maximum(ot[sl][:, None], ot[None, :]), 0.0)
        inter = iw * ih
        union = area[sl][:, None] + area[None, :] - inter
        iou = inter / jnp.maximum(union, 1e-9)
        jgt = ((jax.lax.broadcasted_iota(jnp.int32, (T, _K), 1)) >
               (jax.lax.broadcasted_iota(jnp.int32, (T, _K), 0) + t * T))
        mask_ref[0, sl, :] = ((iou > _NMS_THR) & jgt).astype(jnp.float32)


def _sortiou(cand_scores, cand_idx, cand_labels, bx, by, bw, bh):
    B = cand_scores.shape[0]
    vec = pl.BlockSpec((1, 1, _K), lambda b: (b, 0, 0))
    return pl.pallas_call(
        _sortiou_body,
        grid=(B,),
        in_specs=[vec] * 7,
        out_specs=[
            pl.BlockSpec((1, _K, 4), lambda b: (b, 0, 0)),
            pl.BlockSpec((1, 1, _K), lambda b: (b, 0, 0)),
            pl.BlockSpec((1, 1, _K), lambda b: (b, 0, 0)),
            pl.BlockSpec((1, _K, _K), lambda b: (b, 0, 0)),
        ],
        out_shape=[
            jax.ShapeDtypeStruct((B, _K, 4), jnp.float32),
            jax.ShapeDtypeStruct((B, 1, _K), jnp.int32),
            jax.ShapeDtypeStruct((B, 1, _K), jnp.float32),
            jax.ShapeDtypeStruct((B, _K, _K), jnp.float32),
        ],
        compiler_params=pltpu.CompilerParams(
            dimension_semantics=("arbitrary",),
        ),
    )(cand_scores[:, None, :], cand_idx[:, None, :], cand_labels[:, None, :],
      bx[:, None, :], by[:, None, :], bw[:, None, :], bh[:, None, :])


def _scan_body(s_ref, mask_ref, s2_ref):
    S = s_ref[...]
    keep0 = (S > 0.0).astype(jnp.float32)
    lane = jax.lax.broadcasted_iota(jnp.int32, (1, _K), 1)

    def body(i, keep):
        onehot = (lane == i).astype(jnp.float32)
        keep_i = jnp.sum(keep * onehot, axis=1, keepdims=True)
        mrow = mask_ref[:, i, :]
        return keep * (1.0 - mrow * keep_i)

    keep = jax.lax.fori_loop(0, _PTOPK, body, keep0)
    s2_ref[...] = S * keep


def _scan(s_masked, mask):
    B = s_masked.shape[0]
    return pl.pallas_call(
        _scan_body,
        out_shape=jax.ShapeDtypeStruct((B, _K), jnp.float32),
    )(s_masked, mask)


def kernel(pconf, pcls, pxywh):
    B, N, C = pcls.shape
    pscores, plabels = _scores_labels(pconf, pcls)

    topv, topi = jax.lax.top_k(pscores, _PTOPK)
    bidx = jnp.arange(B)[:, None]
    cand_xywh = pxywh[bidx, topi]
    cand_lab = plabels[bidx, topi]

    pad = _K - _PTOPK
    cand_scores = jnp.concatenate(
        [topv, jnp.full((B, pad), -1.0, jnp.float32)], axis=1)
    cand_idx = jnp.concatenate(
        [topi, jnp.broadcast_to(N + jnp.arange(pad, dtype=jnp.int32), (B, pad))],
        axis=1)
    cand_lab = jnp.concatenate(
        [cand_lab, jnp.zeros((B, pad), jnp.int32)], axis=1)
    cand_xywh = jnp.concatenate(
        [cand_xywh, jnp.zeros((B, pad, 4), jnp.float32)], axis=1)

    b2, l2, s2m, mask = _sortiou(
        cand_scores, cand_idx, cand_lab,
        cand_xywh[..., 0], cand_xywh[..., 1],
        cand_xywh[..., 2], cand_xywh[..., 3])
    s2 = _scan(s2m[:, 0, :], mask)
    l2 = l2[:, 0, :]

    ids_batch2 = jnp.broadcast_to(
        jnp.arange(B, dtype=jnp.int32)[:, None], (B, _PTOPK))
    return (ids_batch2, b2[:, :_PTOPK], l2[:, :_PTOPK], s2[:, :_PTOPK])


# speedup vs baseline: 3.3563x; 1.2091x over previous
import functools

import jax
import jax.numpy as jnp
from jax.experimental import pallas as pl
from jax.experimental.pallas import tpu as pltpu
from jax.experimental.pallas import tpu_sc as plsc

_CONF_THR = 0.05
_PTOPK = 500
_NMS_THR = 0.5
_K = 512


def _score_body(pconf_ref, pcls_ref, ps_ref, plab_ref):
    x = pcls_ref[0]
    s = jax.nn.sigmoid(x)
    m = jnp.max(s, axis=-1, keepdims=True)
    cidx = jax.lax.broadcasted_iota(jnp.int32, s.shape, 1)
    lab = jnp.min(jnp.where(s == m, cidx, jnp.int32(2**30)), axis=-1,
                  keepdims=True)
    conf = jax.nn.sigmoid(pconf_ref[0])
    ps_ref[0] = conf * m
    plab_ref[0] = lab


def _scores_labels(pconf, pcls):
    B, N, C = pcls.shape
    CHUNK = 2000
    grid = (B, N // CHUNK)
    ps, plab = pl.pallas_call(
        _score_body,
        grid=grid,
        in_specs=[
            pl.BlockSpec((1, CHUNK, 1), lambda b, c: (b, c, 0)),
            pl.BlockSpec((1, CHUNK, C), lambda b, c: (b, c, 0)),
        ],
        out_specs=[
            pl.BlockSpec((1, CHUNK, 1), lambda b, c: (b, c, 0)),
            pl.BlockSpec((1, CHUNK, 1), lambda b, c: (b, c, 0)),
        ],
        out_shape=[
            jax.ShapeDtypeStruct((B, N, 1), jnp.float32),
            jax.ShapeDtypeStruct((B, N, 1), jnp.int32),
        ],
        compiler_params=pltpu.CompilerParams(
            dimension_semantics=("arbitrary", "arbitrary"),
        ),
    )(pconf[..., None], pcls)
    return ps[..., 0], plab[..., 0]


_N = 20000
_NCHUNK = _N // 16
_SELPAD = _K + 16


def _sc_select_body(ps_hbm, plab_hbm, pxy_hbm,
                    so_hbm, io_hbm, lo_hbm, xo_hbm,
                    sv, lv, hist, ssel, isel, lsel, xysel, xyv,
                    sem1, sem2, sem3):
    core = jax.lax.axis_index("c")
    b = jax.lax.axis_index("s")

    @pl.when(core == 0)
    def _():
        iota = jax.lax.iota(jnp.int32, 16)
        ones = jnp.ones((16,), jnp.int32)

        h_s = pltpu.async_copy(ps_hbm.at[b], sv, sem1)
        h_l = pltpu.async_copy(plab_hbm.at[b], lv, sem2)
        h_x = pltpu.async_copy(pxy_hbm.at[b], xyv, sem3)
        h_s.wait()

        quota = jnp.int32(_PTOPK)
        prefix = jnp.int32(0)
        levels = ((None, 22, 0xFF), (22, 14, 0xFF),
                  (14, 6, 0xFF), (6, 0, 0x3F))
        for hs, ns, bmask in levels:
            for c in range(256):
                hist[pl.ds(c * 16, 16)] = jnp.zeros((16,), jnp.int32)

            pfx = prefix

            def scan_body(i, _, hs=hs, ns=ns, bmask=bmask, pfx=pfx):
                x = sv[pl.ds(i * 16, 16)]
                bts = jax.lax.bitcast_convert_type(x, jnp.int32)
                bucket = jnp.right_shift(bts, ns) & bmask
                flat = iota * 256 + bucket
                if hs is None:
                    plsc.addupdate_scatter(hist, [flat], ones)
                else:
                    match = jnp.right_shift(bts, hs) == pfx
                    plsc.addupdate_scatter(hist, [flat], ones, mask=match)
                return 0

            jax.lax.fori_loop(0, _NCHUNK, scan_body, 0)

            running = jnp.int32(0)
            dstar = jnp.int32(0)
            above = jnp.int32(0)
            for c in range(15, -1, -1):
                m = hist[pl.ds(c * 16, 16)]
                for r in range(1, 16):
                    m = m + hist[pl.ds(r * 256 + c * 16, 16)]
                rv = jnp.flip(m)
                cs = jnp.cumsum(rv)
                suffix = running + cs
                cond = (suffix >= quota) & ((suffix - rv) < quota)
                ids_rev = (c * 16 + 15) - iota
                dstar = dstar + jnp.sum(jnp.where(cond, ids_rev, 0))
                above = above + jnp.sum(jnp.where(cond, suffix - rv, 0))
                running = running + jnp.sum(m)
            quota = quota - above
            if ns == 0:
                thr = jnp.left_shift(prefix, 6) | dstar
            else:
                prefix = jnp.left_shift(prefix, 8) | dstar
        tie_quota = quota

        for c in range(_SELPAD // 16):
            ssel[pl.ds(c * 16, 16)] = jnp.full((16,), -1.0, jnp.float32)
            isel[pl.ds(c * 16, 16)] = _N + c * 16 + iota

        def compact_body(i, carry):
            woff, ties = carry
            x = sv[pl.ds(i * 16, 16)]
            bts = jax.lax.bitcast_convert_type(x, jnp.int32)
            gt = bts > thr
            eq = bts == thr
            pref = jnp.cumsum(eq.astype(jnp.int32))
            keep = gt | (eq & ((ties + pref) <= tie_quota))
            idxv = i * 16 + iota
            plsc.store_compressed(ssel.at[pl.ds(woff, 16)], x, mask=keep)
            plsc.store_compressed(isel.at[pl.ds(woff, 16)], idxv, mask=keep)
            return (woff + jnp.sum(keep.astype(jnp.int32)),
                    ties + jnp.sum(eq.astype(jnp.int32)))

        jax.lax.fori_loop(0, _NCHUNK, compact_body,
                          (jnp.int32(0), jnp.int32(0)))

        h_l.wait()
        h_x.wait()
        for c in range(_K // 16):
            v = isel[pl.ds(c * 16, 16)]
            vc = jnp.minimum(v, _N - 1)
            lsel[pl.ds(c * 16, 16)] = plsc.load_gather(lv, [vc])
            vc4 = vc * 4
            for c4 in range(4):
                col = plsc.load_gather(xyv, [vc4 + c4])
                plsc.store_scatter(xysel, [iota * 4 + (c * 64 + c4)], col)

        pltpu.sync_copy(ssel.at[pl.ds(0, _K)], so_hbm.at[b])
        pltpu.sync_copy(isel.at[pl.ds(0, _K)], io_hbm.at[b])
        pltpu.sync_copy(lsel, lo_hbm.at[b])
        pltpu.sync_copy(xysel, xo_hbm.at[b])


def _sc_select(pscores, plabels, pxywh):
    B, N = pscores.shape
    sel = pl.kernel(
        _sc_select_body,
        out_type=[
            jax.ShapeDtypeStruct((B, _K), jnp.float32),
            jax.ShapeDtypeStruct((B, _K), jnp.int32),
            jax.ShapeDtypeStruct((B, _K), jnp.int32),
            jax.ShapeDtypeStruct((B, 4 * _K), jnp.float32),
        ],
        mesh=plsc.VectorSubcoreMesh(core_axis_name="c", subcore_axis_name="s"),
        compiler_params=pltpu.CompilerParams(needs_layout_passes=False, use_tc_tiling_on_sc=False),
        scratch_types=[
            pltpu.VMEM((N,), jnp.float32),
            pltpu.VMEM((N,), jnp.int32),
            pltpu.VMEM((4096,), jnp.int32),
            pltpu.VMEM((_SELPAD,), jnp.float32),
            pltpu.VMEM((_SELPAD,), jnp.int32),
            pltpu.VMEM((_K,), jnp.int32),
            pltpu.VMEM((4 * _K,), jnp.float32),
            pltpu.VMEM((4 * N,), jnp.float32),
            pltpu.SemaphoreType.DMA,
            pltpu.SemaphoreType.DMA,
            pltpu.SemaphoreType.DMA,
        ],
    )
    so, io, lo, xo = sel(pscores, plabels, pxywh.reshape(B, 4 * N))
    return so, io, lo, xo.reshape(B, _K, 4)


def _sortiou_body(scores_ref, idx_ref, lab_ref,
                  bx_ref, by_ref, bw_ref, bh_ref,
                  b2_ref, l2_ref, s2m_ref, mask_ref):
    s = scores_ref[0, 0]
    ix = idx_ref[0, 0]
    lb = lab_ref[0, 0]

    T = 64
    nt = _K // T

    ranks = []
    for t in range(nt):
        si = s[t * T:(t + 1) * T][:, None]
        ixi = ix[t * T:(t + 1) * T][:, None]
        before = ((s[None, :] > si) |
                  ((s[None, :] == si) & (ix[None, :] < ixi)))
        ranks.append(jnp.sum(before.astype(jnp.int32), axis=1))
    rank = jnp.concatenate(ranks)

    cols = (bx_ref[0, 0], by_ref[0, 0], bw_ref[0, 0], bh_ref[0, 0], s)
    parts = [[] for _ in range(6)]
    for t in range(nt):
        ot_iota = jax.lax.broadcasted_iota(jnp.int32, (T, _K), 0) + t * T
        oh = (rank[None, :] == ot_iota)
        for ci, col in enumerate(cols):
            parts[ci].append(
                jnp.max(jnp.where(oh, col[None, :], jnp.float32(-3e38)),
                        axis=1))
        parts[5].append(
            jnp.max(jnp.where(oh, lb[None, :], jnp.int32(-2**31 + 1)),
                    axis=1))
    cx, cy, w, h, so = (jnp.concatenate(p) for p in parts[:5])
    lo = jnp.concatenate(parts[5])

    so = jnp.where(so > _CONF_THR, so, jnp.float32(0.0))

    xl = cx - w * 0.5
    yt = cy - h * 0.5
    xr = cx + w * 0.5
    yb = cy + h * 0.5

    b2_ref[0] = jnp.stack([xl, yt, xr, yb], axis=1)
    l2_ref[0, 0] = lo
    s2m_ref[0, 0] = so

    off = lo.astype(jnp.float32) * 4.0
    ol = xl + off
    ot = yt + off
    orr = xr + off
    ob = yb + off
    area = jnp.clip(orr - ol, 0.0) * jnp.clip(ob - ot, 0.0)
    for t in range(nt):
        sl = slice(t * T, (t + 1) * T)
        iw = jnp.clip(jnp.minimum(orr[sl][:, None], orr[None, :]) -
                      jnp.maximum(ol[sl][:, None], ol[None, :]), 0.0)
        ih = jnp.clip(jnp.minimum(ob[sl][:, None], ob[None, :]) -
                      jnp.maximum(ot[sl][:, None], ot[None, :]), 0.0)
        inter = iw * ih
        union = area[sl][:, None] + area[None, :] - inter
        iou = inter / jnp.maximum(union, 1e-9)
        jgt = ((jax.lax.broadcasted_iota(jnp.int32, (T, _K), 1)) >
               (jax.lax.broadcasted_iota(jnp.int32, (T, _K), 0) + t * T))
        mask_ref[0, sl, :] = ((iou > _NMS_THR) & jgt).astype(jnp.float32)


def _sortiou(cand_scores, cand_idx, cand_labels, bx, by, bw, bh):
    B = cand_scores.shape[0]
    vec = pl.BlockSpec((1, 1, _K), lambda b: (b, 0, 0))
    return pl.pallas_call(
        _sortiou_body,
        grid=(B,),
        in_specs=[vec] * 7,
        out_specs=[
            pl.BlockSpec((1, _K, 4), lambda b: (b, 0, 0)),
            pl.BlockSpec((1, 1, _K), lambda b: (b, 0, 0)),
            pl.BlockSpec((1, 1, _K), lambda b: (b, 0, 0)),
            pl.BlockSpec((1, _K, _K), lambda b: (b, 0, 0)),
        ],
        out_shape=[
            jax.ShapeDtypeStruct((B, _K, 4), jnp.float32),
            jax.ShapeDtypeStruct((B, 1, _K), jnp.int32),
            jax.ShapeDtypeStruct((B, 1, _K), jnp.float32),
            jax.ShapeDtypeStruct((B, _K, _K), jnp.float32),
        ],
        compiler_params=pltpu.CompilerParams(
            dimension_semantics=("arbitrary",),
        ),
    )(cand_scores[:, None, :], cand_idx[:, None, :], cand_labels[:, None, :],
      bx[:, None, :], by[:, None, :], bw[:, None, :], bh[:, None, :])


def _scan_body(s_ref, mask_ref, s2_ref):
    S = s_ref[...]
    keep0 = (S > 0.0).astype(jnp.float32)
    lane = jax.lax.broadcasted_iota(jnp.int32, (1, _K), 1)

    def body(i, keep):
        onehot = (lane == i).astype(jnp.float32)
        keep_i = jnp.sum(keep * onehot, axis=1, keepdims=True)
        mrow = mask_ref[:, i, :]
        return keep * (1.0 - mrow * keep_i)

    keep = jax.lax.fori_loop(0, _PTOPK, body, keep0)
    s2_ref[...] = S * keep


def _scan(s_masked, mask):
    B = s_masked.shape[0]
    return pl.pallas_call(
        _scan_body,
        out_shape=jax.ShapeDtypeStruct((B, _K), jnp.float32),
    )(s_masked, mask)


def kernel(pconf, pcls, pxywh):
    B, N, C = pcls.shape
    pscores, plabels = _scores_labels(pconf, pcls)

    cand_scores, cand_idx, cand_lab, cand_xywh = _sc_select(
        pscores, plabels, pxywh)

    b2, l2, s2m, mask = _sortiou(
        cand_scores, cand_idx, cand_lab,
        cand_xywh[..., 0], cand_xywh[..., 1],
        cand_xywh[..., 2], cand_xywh[..., 3])
    s2 = _scan(s2m[:, 0, :], mask)
    l2 = l2[:, 0, :]

    ids_batch2 = jnp.broadcast_to(
        jnp.arange(B, dtype=jnp.int32)[:, None], (B, _PTOPK))
    return (ids_batch2, b2[:, :_PTOPK], l2[:, :_PTOPK], s2[:, :_PTOPK])
